# baseline (speedup 1.0000x reference)
import functools

import jax
import jax.numpy as jnp
from jax import lax
from jax.experimental import pallas as pl
from jax.experimental.pallas import tpu as pltpu
from jax.experimental.pallas import tpu_sc as plsc

_NC = 2
_NS = 16
_NW = _NC * _NS

_CHUNK = 1664


def _sc_gather(table, idx):
    n = idx.shape[0]
    d = table.shape[1]
    per_w = n // _NW
    n_chunks = per_w // _CHUNK
    n_pairs = n_chunks // 2

    mesh = plsc.VectorSubcoreMesh(core_axis_name="c", subcore_axis_name="s")

    @functools.partial(
        pl.kernel,
        mesh=mesh,
        out_type=jax.ShapeDtypeStruct((n, d), jnp.float32),
        scratch_types=[
            pltpu.VMEM((_CHUNK,), jnp.int32),
            pltpu.VMEM((_CHUNK,), jnp.int32),
            pltpu.VMEM((_CHUNK, d), jnp.float32),
            pltpu.VMEM((_CHUNK, d), jnp.float32),
            pltpu.SemaphoreType.DMA,
            pltpu.SemaphoreType.DMA,
            pltpu.SemaphoreType.DMA,
            pltpu.SemaphoreType.DMA,
        ],
        compiler_params=pltpu.CompilerParams(use_tc_tiling_on_sc=False),
    )
    def gather_kernel(table_hbm, idx_hbm, out_hbm,
                      idx0, idx1, rows0, rows1, sg0, sg1, sw0, sw1):
        wid = lax.axis_index("s") * _NC + lax.axis_index("c")
        base = wid * per_w

        def start_gather(i, idx_v, rows_v, sem):
            off = base + i * _CHUNK
            pltpu.sync_copy(idx_hbm.at[pl.ds(off, _CHUNK)], idx_v)
            pltpu.async_copy(table_hbm.at[idx_v], rows_v, sem)

        def drain_gather(rows_v, sem):
            pltpu.make_async_copy(table_hbm.at[pl.ds(0, _CHUNK)], rows_v, sem).wait()

        def start_write(i, rows_v, sem):
            pltpu.async_copy(rows_v, out_hbm.at[pl.ds(base + i * _CHUNK, _CHUNK)], sem)

        def drain_write(rows_v, sem):
            pltpu.make_async_copy(rows_v, out_hbm.at[pl.ds(base, _CHUNK)], sem).wait()

        start_gather(0, idx0, rows0, sg0)

        def body(j, carry):
            a = 2 * j
            start_gather(a + 1, idx1, rows1, sg1)
            drain_gather(rows0, sg0)
            start_write(a, rows0, sw0)
            @pl.when(j < n_pairs - 1)
            def _():
                drain_write(rows0, sw0)
                start_gather(a + 2, idx0, rows0, sg0)
            drain_gather(rows1, sg1)
            start_write(a + 1, rows1, sw1)
            drain_write(rows1, sw1)
            return carry

        lax.fori_loop(0, n_pairs, body, 0)
        drain_write(rows0, sw0)

    return gather_kernel(table, idx)


def _tc_dense(x, w, b):
    bsz, k = x.shape
    m = w.shape[1]
    bm = 1024

    def dense_kernel(x_ref, w_ref, b_ref, o_ref):
        acc = jnp.dot(x_ref[...], w_ref[...], preferred_element_type=jnp.float32)
        o_ref[...] = jnp.maximum(acc + b_ref[...], 0.0)

    return pl.pallas_call(
        dense_kernel,
        grid=(bsz // bm,),
        in_specs=[
            pl.BlockSpec((bm, k), lambda i: (i, 0)),
            pl.BlockSpec((k, m), lambda i: (0, 0)),
            pl.BlockSpec((1, m), lambda i: (0, 0)),
        ],
        out_specs=pl.BlockSpec((bm, m), lambda i: (i, 0)),
        out_shape=jax.ShapeDtypeStruct((bsz, m), jnp.float32),
    )(x, w, b)


def kernel(input, table, W, b):
    bsz, f = input.shape
    d = table.shape[1]
    idx = lax.optimization_barrier(input.reshape(-1).astype(jnp.int32))
    gathered = _sc_gather(table, idx)
    x = gathered.reshape(bsz, f * d)
    return _tc_dense(x, W, b.reshape(1, -1))
